# Initial kernel scaffold; baseline (speedup 1.0000x reference)
import jax
import jax.numpy as jnp
from jax.experimental import pallas as pl


def kernel(activations, anchor_idx, pos_idx, neg_idx):
    raise NotImplementedError("write your pallas kernel here")


# speedup vs baseline: 5.6459x; 5.6459x over previous
import functools

import jax
import jax.numpy as jnp
from jax import lax
from jax.experimental import pallas as pl
from jax.experimental.pallas import tpu as pltpu
from jax.experimental.pallas import tpu_sc as plsc

TEMPERATURE = 0.1
EPS = 1e-8

_NC = 2
_NS = 16
_NW = _NC * _NS

_ROW_W = 16
_Q = 128


def _normalize_body(x_ref, o_ref):
    x = x_ref[...]
    nrm = jnp.sqrt(jnp.sum(x * x, axis=-1, keepdims=True))
    o_ref[...] = (x / jnp.maximum(nrm, EPS)).astype(jnp.bfloat16)


def _normalize(flat):
    L, R, D = flat.shape
    rb = 1024
    return pl.pallas_call(
        _normalize_body,
        grid=(L, R // rb),
        in_specs=[pl.BlockSpec((1, rb, D), lambda l, i: (l, i, 0))],
        out_specs=pl.BlockSpec((1, rb, D), lambda l, i: (l, i, 0)),
        out_shape=jax.ShapeDtypeStruct((L, R, D), jnp.bfloat16),
    )(flat)


def _gram_body(a_ref, b_ref, o_ref):
    o_ref[...] = lax.dot_general(
        a_ref[0], b_ref[0],
        (((1,), (1,)), ((), ())),
        preferred_element_type=jnp.float32,
    )[None]


def _gram(xhat):
    L, R, D = xhat.shape
    tb = 512
    nt = R // tb
    return pl.pallas_call(
        _gram_body,
        grid=(L, nt, nt),
        in_specs=[
            pl.BlockSpec((1, tb, D), lambda l, i, j: (l, i, 0)),
            pl.BlockSpec((1, tb, D), lambda l, i, j: (l, j, 0)),
        ],
        out_specs=pl.BlockSpec((1, tb, tb), lambda l, i, j: (l, i, j)),
        out_shape=jax.ShapeDtypeStruct((L, R, R), jnp.float32),
    )(xhat, xhat)


def _make_sc_gather(L, R, N, NB):
    words_per_layer = R * R
    ch = N // _NW
    nq = ch // _Q

    mesh = plsc.VectorSubcoreMesh(core_axis_name="c", subcore_axis_name="s")

    @functools.partial(
        pl.kernel,
        out_type=jax.ShapeDtypeStruct((L, NB, N), jnp.float32),
        mesh=mesh,
        scratch_types=[
            pltpu.VMEM((ch,), jnp.int32),
            pltpu.VMEM((NB, ch), jnp.int32),
            pltpu.VMEM((_Q,), jnp.int32),
            pltpu.VMEM((ch,), jnp.float32),
            pltpu.SemaphoreType.DMA,
        ],
    )
    def sc_gather(g_hbm, a_hbm, b_hbm, out_hbm, a_v, b_v, idx_v, sim_v, sem):
        wid = lax.axis_index("s") * _NC + lax.axis_index("c")
        base = wid * ch
        pltpu.sync_copy(a_hbm.at[pl.ds(base, ch)], a_v)
        for r in range(NB):
            pltpu.sync_copy(b_hbm.at[r, pl.ds(base, ch)], b_v.at[r])

        def layer_body(l, carry):
            layer_off = l * words_per_layer
            for r in range(NB):
                def q_body(q, carry2):
                    qb = q * _Q

                    def idx_body(c, carry3):
                        off = qb + c * 16
                        a16 = a_v[pl.ds(off, 16)]
                        b16 = b_v[r, pl.ds(off, 16)]
                        idx_v[pl.ds(c * 16, 16)] = layer_off + a16 * R + b16
                        return carry3

                    lax.fori_loop(0, _Q // 16, idx_body, 0)
                    pltpu.async_copy(
                        g_hbm.at[idx_v], sim_v.at[pl.ds(qb, _Q)], sem).wait()
                    return carry2

                lax.fori_loop(0, nq, q_body, 0)
                pltpu.sync_copy(sim_v, out_hbm.at[l, r, pl.ds(base, ch)])
            return carry

        lax.fori_loop(0, L, layer_body, 0)

    return sc_gather


def _loss_body(s_ref, o_ref):
    logits = s_ref[...] * (1.0 / TEMPERATURE)
    pos = logits[:, 0, :]
    denom = jnp.exp(pos) + jnp.sum(jnp.exp(logits[:, 1:, :]), axis=1)
    terms = pos - jnp.log(denom)
    L = terms.shape[0]
    o_ref[...] = (-jnp.sum(jnp.mean(terms, axis=-1)) / L).reshape(1, 1)


def _loss(sims):
    return pl.pallas_call(
        _loss_body,
        out_shape=jax.ShapeDtypeStruct((1, 1), jnp.float32),
    )(sims)


def kernel(activations, anchor_idx, pos_idx, neg_idx):
    L, B, S, D = activations.shape
    R = B * S
    N, K = neg_idx.shape
    flat = activations.reshape(L, R, D)

    xhat = _normalize(flat)
    gram = _gram(xhat)
    g_flat = gram.reshape(L * R * R)

    b_all = jnp.concatenate([pos_idx[None, :], neg_idx.T], axis=0)
    sims = _make_sc_gather(L, R, N, K + 1)(g_flat, anchor_idx, b_all)
    return _loss(sims)[0, 0]


# speedup vs baseline: 8.1359x; 1.4410x over previous
import functools

import jax
import jax.numpy as jnp
from jax import lax
from jax.experimental import pallas as pl
from jax.experimental.pallas import tpu as pltpu
from jax.experimental.pallas import tpu_sc as plsc

TEMPERATURE = 0.1
EPS = 1e-8

_NC = 2
_NS = 16
_NW = _NC * _NS

_ROW_W = 16
_Q = 128


def _normalize_body(x_ref, o_ref):
    x = x_ref[...]
    nrm = jnp.sqrt(jnp.sum(x * x, axis=-1, keepdims=True))
    o_ref[...] = (x / jnp.maximum(nrm, EPS)).astype(jnp.bfloat16)


def _normalize(flat):
    L, R, D = flat.shape
    rb = 1024
    return pl.pallas_call(
        _normalize_body,
        grid=(L, R // rb),
        in_specs=[pl.BlockSpec((1, rb, D), lambda l, i: (l, i, 0))],
        out_specs=pl.BlockSpec((1, rb, D), lambda l, i: (l, i, 0)),
        out_shape=jax.ShapeDtypeStruct((L, R, D), jnp.bfloat16),
    )(flat)


_TB_I = 2048
_TB_J = 128


def _gram_body(a_ref, b_ref, o_ref):
    o_ref[...] = lax.dot_general(
        a_ref[0], b_ref[0],
        (((1,), (1,)), ((), ())),
        preferred_element_type=jnp.float32,
    )


def _gram(xhat):
    L, R, D = xhat.shape
    ni, nj = R // _TB_I, R // _TB_J
    return pl.pallas_call(
        _gram_body,
        grid=(L, ni, nj),
        in_specs=[
            pl.BlockSpec((1, _TB_I, D), lambda l, i, j: (l, i, 0)),
            pl.BlockSpec((1, _TB_J, D), lambda l, i, j: (l, j, 0)),
        ],
        out_specs=pl.BlockSpec(
            (_TB_I, _TB_J), lambda l, i, j: ((l * ni + i) * nj + j, 0)),
        out_shape=jax.ShapeDtypeStruct((L * ni * nj * _TB_I, _TB_J),
                                       jnp.float32),
    )(xhat, xhat)


def _make_sc_gather(L, R, N, NB):
    ni, nj = R // _TB_I, R // _TB_J
    sh_i = _TB_I.bit_length() - 1
    sh_j = _TB_J.bit_length() - 1
    block_words = _TB_I * _TB_J
    ch = N // _NW
    nq = ch // _Q

    mesh = plsc.VectorSubcoreMesh(core_axis_name="c", subcore_axis_name="s")

    @functools.partial(
        pl.kernel,
        out_type=jax.ShapeDtypeStruct((L, NB, N), jnp.float32),
        mesh=mesh,
        scratch_types=[
            pltpu.VMEM((ch,), jnp.int32),
            pltpu.VMEM((NB, ch), jnp.int32),
            pltpu.VMEM((_Q,), jnp.int32),
            pltpu.VMEM((ch,), jnp.float32),
            pltpu.SemaphoreType.DMA,
        ],
    )
    def sc_gather(g_hbm, a_hbm, b_hbm, out_hbm, a_v, b_v, idx_v, sim_v, sem):
        wid = lax.axis_index("s") * _NC + lax.axis_index("c")
        base = wid * ch
        pltpu.sync_copy(a_hbm.at[pl.ds(base, ch)], a_v)
        for r in range(NB):
            pltpu.sync_copy(b_hbm.at[r, pl.ds(base, ch)], b_v.at[r])

        def layer_body(l, carry):
            lni = l * ni
            for r in range(NB):
                def q_body(q, carry2):
                    qb = q * _Q

                    def idx_body(c, carry3):
                        off = qb + c * 16
                        a16 = a_v[pl.ds(off, 16)]
                        b16 = b_v[r, pl.ds(off, 16)]
                        blk = ((lni + lax.shift_right_logical(a16, sh_i))
                               * nj + lax.shift_right_logical(b16, sh_j))
                        idx_v[pl.ds(c * 16, 16)] = (
                            blk * block_words
                            + lax.bitwise_and(a16, _TB_I - 1) * _TB_J
                            + lax.bitwise_and(b16, _TB_J - 1))
                        return carry3

                    lax.fori_loop(0, _Q // 16, idx_body, 0)
                    pltpu.async_copy(
                        g_hbm.at[idx_v], sim_v.at[pl.ds(qb, _Q)], sem).wait()
                    return carry2

                lax.fori_loop(0, nq, q_body, 0)
                pltpu.sync_copy(sim_v, out_hbm.at[l, r, pl.ds(base, ch)])
            return carry

        lax.fori_loop(0, L, layer_body, 0)

    return sc_gather


def _loss_body(s_ref, o_ref):
    logits = s_ref[...] * (1.0 / TEMPERATURE)
    pos = logits[:, 0, :]
    denom = jnp.exp(pos) + jnp.sum(jnp.exp(logits[:, 1:, :]), axis=1)
    terms = pos - jnp.log(denom)
    L = terms.shape[0]
    o_ref[...] = (-jnp.sum(jnp.mean(terms, axis=-1)) / L).reshape(1, 1)


def _loss(sims):
    return pl.pallas_call(
        _loss_body,
        out_shape=jax.ShapeDtypeStruct((1, 1), jnp.float32),
    )(sims)


def kernel(activations, anchor_idx, pos_idx, neg_idx):
    L, B, S, D = activations.shape
    R = B * S
    N, K = neg_idx.shape
    flat = activations.reshape(L, R, D)

    xhat = _normalize(flat)
    gram = _gram(xhat)
    g_flat = gram.reshape(L * R * R)

    b_all = jnp.concatenate([pos_idx[None, :], neg_idx.T], axis=0)
    sims = _make_sc_gather(L, R, N, K + 1)(g_flat, anchor_idx, b_all)
    return _loss(sims)[0, 0]


# speedup vs baseline: 9.4288x; 1.1589x over previous
import functools

import jax
import jax.numpy as jnp
from jax import lax
from jax.experimental import pallas as pl
from jax.experimental.pallas import tpu as pltpu
from jax.experimental.pallas import tpu_sc as plsc

TEMPERATURE = 0.1
EPS = 1e-8

_NC = 2
_NS = 16
_NW = _NC * _NS

_ROW_W = 16
_Q = 128


def _normalize_body(x_ref, o_ref):
    x = x_ref[...]
    nrm = jnp.sqrt(jnp.sum(x * x, axis=-1, keepdims=True))
    o_ref[...] = (x / jnp.maximum(nrm, EPS)).astype(jnp.bfloat16)


def _normalize(flat):
    L, R, D = flat.shape
    rb = 1024
    return pl.pallas_call(
        _normalize_body,
        grid=(L, R // rb),
        in_specs=[pl.BlockSpec((1, rb, D), lambda l, i: (l, i, 0))],
        out_specs=pl.BlockSpec((1, rb, D), lambda l, i: (l, i, 0)),
        out_shape=jax.ShapeDtypeStruct((L, R, D), jnp.bfloat16),
    )(flat)


_TB_I = 2048
_TB_J = 128


def _gram_body(a_ref, b_ref, o_ref):
    o_ref[...] = lax.dot_general(
        a_ref[0], b_ref[0],
        (((1,), (1,)), ((), ())),
        preferred_element_type=jnp.float32,
    )


def _gram(xhat):
    L, R, D = xhat.shape
    ni, nj = R // _TB_I, R // _TB_J
    return pl.pallas_call(
        _gram_body,
        grid=(L, ni, nj),
        in_specs=[
            pl.BlockSpec((1, _TB_I, D), lambda l, i, j: (l, i, 0)),
            pl.BlockSpec((1, _TB_J, D), lambda l, i, j: (l, j, 0)),
        ],
        out_specs=pl.BlockSpec(
            (_TB_I, _TB_J), lambda l, i, j: ((l * ni + i) * nj + j, 0)),
        out_shape=jax.ShapeDtypeStruct((L * ni * nj * _TB_I, _TB_J),
                                       jnp.float32),
    )(xhat, xhat)


def _make_sc_gather(L, R, N, NB):
    ni, nj = R // _TB_I, R // _TB_J
    sh_i = _TB_I.bit_length() - 1
    sh_j = _TB_J.bit_length() - 1
    block_words = _TB_I * _TB_J
    ch = N // _NW
    nq = ch // _Q

    mesh = plsc.VectorSubcoreMesh(core_axis_name="c", subcore_axis_name="s")

    @functools.partial(
        pl.kernel,
        out_type=jax.ShapeDtypeStruct((L, NB, N), jnp.float32),
        mesh=mesh,
        scratch_types=[
            pltpu.VMEM((ch,), jnp.int32),
            pltpu.VMEM((NB, ch), jnp.int32),
            pltpu.VMEM((NB * nq, _Q), jnp.int32),
            pltpu.VMEM((NB, ch), jnp.float32),
            pltpu.SemaphoreType.DMA,
        ],
    )
    def sc_gather(g_hbm, a_hbm, b_hbm, out_hbm, a_v, b_v, idx_v, sim_v, sem):
        wid = lax.axis_index("s") * _NC + lax.axis_index("c")
        base = wid * ch
        pltpu.sync_copy(a_hbm.at[pl.ds(base, ch)], a_v)
        for r in range(NB):
            pltpu.sync_copy(b_hbm.at[r, pl.ds(base, ch)], b_v.at[r])

        def layer_body(l, carry):
            lni = l * ni
            for r in range(NB):
                def idx_body(c, carry3):
                    off = c * 16
                    a16 = a_v[pl.ds(off, 16)]
                    b16 = b_v[r, pl.ds(off, 16)]
                    blk = ((lni + lax.shift_right_logical(a16, sh_i))
                           * nj + lax.shift_right_logical(b16, sh_j))
                    word = (blk * block_words
                            + lax.bitwise_and(a16, _TB_I - 1) * _TB_J
                            + lax.bitwise_and(b16, _TB_J - 1))
                    q = lax.div(c, jnp.int32(_Q // 16))
                    within = lax.rem(c, jnp.int32(_Q // 16))
                    idx_v[r * nq + q, pl.ds(within * 16, 16)] = word
                    return carry3

                lax.fori_loop(0, ch // 16, idx_body, 0)
            copies = []
            for r in range(NB):
                for q in range(nq):
                    copies.append(pltpu.async_copy(
                        g_hbm.at[idx_v.at[r * nq + q]],
                        sim_v.at[r, pl.ds(q * _Q, _Q)], sem))
            for cp in copies:
                cp.wait()
            pltpu.sync_copy(sim_v, out_hbm.at[l, :, pl.ds(base, ch)])
            return carry

        lax.fori_loop(0, L, layer_body, 0)

    return sc_gather


def _loss_body(s_ref, o_ref):
    logits = s_ref[...] * (1.0 / TEMPERATURE)
    pos = logits[:, 0, :]
    denom = jnp.exp(pos) + jnp.sum(jnp.exp(logits[:, 1:, :]), axis=1)
    terms = pos - jnp.log(denom)
    L = terms.shape[0]
    o_ref[...] = (-jnp.sum(jnp.mean(terms, axis=-1)) / L).reshape(1, 1)


def _loss(sims):
    return pl.pallas_call(
        _loss_body,
        out_shape=jax.ShapeDtypeStruct((1, 1), jnp.float32),
    )(sims)


def kernel(activations, anchor_idx, pos_idx, neg_idx):
    L, B, S, D = activations.shape
    R = B * S
    N, K = neg_idx.shape
    flat = activations.reshape(L, R, D)

    xhat = _normalize(flat)
    gram = _gram(xhat)
    g_flat = gram.reshape(L * R * R)

    b_all = jnp.concatenate([pos_idx[None, :], neg_idx.T], axis=0)
    sims = _make_sc_gather(L, R, N, K + 1)(g_flat, anchor_idx, b_all)
    return _loss(sims)[0, 0]


# speedup vs baseline: 14.5531x; 1.5435x over previous
import functools

import jax
import jax.numpy as jnp
from jax import lax
from jax.experimental import pallas as pl
from jax.experimental.pallas import tpu as pltpu
from jax.experimental.pallas import tpu_sc as plsc

TEMPERATURE = 0.1
EPS = 1e-8

_NC = 2
_NS = 16
_NW = _NC * _NS

_ROW_W = 16
_Q = 128


def _normalize_body(x_ref, o_ref):
    x = x_ref[...]
    nrm = jnp.sqrt(jnp.sum(x * x, axis=-1, keepdims=True))
    o_ref[...] = (x / jnp.maximum(nrm, EPS)).astype(jnp.bfloat16)


def _normalize(flat):
    L, R, D = flat.shape
    rb = 1024
    return pl.pallas_call(
        _normalize_body,
        grid=(L, R // rb),
        in_specs=[pl.BlockSpec((1, rb, D), lambda l, i: (l, i, 0))],
        out_specs=pl.BlockSpec((1, rb, D), lambda l, i: (l, i, 0)),
        out_shape=jax.ShapeDtypeStruct((L, R, D), jnp.bfloat16),
    )(flat)


_TB_I = 2048
_TB_J = 256


def _gram_body(a_ref, b_ref, o_ref):
    d = lax.dot_general(
        a_ref[0], b_ref[0],
        (((1,), (1,)), ((), ())),
        preferred_element_type=jnp.float32,
    )
    lo = lax.bitcast_convert_type(
        d[:, :_TB_J // 2].astype(jnp.bfloat16), jnp.uint16)
    hi = lax.bitcast_convert_type(
        d[:, _TB_J // 2:].astype(jnp.bfloat16), jnp.uint16)
    o_ref[...] = lo.astype(jnp.uint32) | (hi.astype(jnp.uint32) << 16)


def _gram(xhat):
    L, R, D = xhat.shape
    ni, nj = R // _TB_I, R // _TB_J
    return pl.pallas_call(
        _gram_body,
        grid=(L, ni, nj),
        in_specs=[
            pl.BlockSpec((1, _TB_I, D), lambda l, i, j: (l, i, 0)),
            pl.BlockSpec((1, _TB_J, D), lambda l, i, j: (l, j, 0)),
        ],
        out_specs=pl.BlockSpec(
            (_TB_I, _TB_J // 2), lambda l, i, j: ((l * ni + i) * nj + j, 0)),
        out_shape=jax.ShapeDtypeStruct((L * ni * nj * _TB_I, _TB_J // 2),
                                       jnp.uint32),
    )(xhat, xhat)


def _make_sc_gather(L, R, N, NB):
    ni, nj = R // _TB_I, R // _TB_J
    sh_i = _TB_I.bit_length() - 1
    sh_j = _TB_J.bit_length() - 1
    hw = _TB_J // 2
    block_words = _TB_I * hw
    ch = N // _NW
    nq = ch // _Q

    mesh = plsc.VectorSubcoreMesh(core_axis_name="c", subcore_axis_name="s")

    @functools.partial(
        pl.kernel,
        out_type=jax.ShapeDtypeStruct((L, NB, N), jnp.uint32),
        mesh=mesh,
        scratch_types=[
            pltpu.VMEM((ch,), jnp.int32),
            pltpu.VMEM((NB, ch), jnp.int32),
            pltpu.VMEM((NB * nq, _Q), jnp.int32),
            pltpu.VMEM((NB, ch), jnp.uint32),
            pltpu.SemaphoreType.DMA,
        ],
    )
    def sc_gather(g_hbm, a_hbm, b_hbm, out_hbm, a_v, b_v, idx_v, simu_v,
                  sem):
        wid = lax.axis_index("s") * _NC + lax.axis_index("c")
        base = wid * ch
        pltpu.sync_copy(a_hbm.at[pl.ds(base, ch)], a_v)
        for r in range(NB):
            pltpu.sync_copy(b_hbm.at[r, pl.ds(base, ch)], b_v.at[r])

        def layer_body(l, carry):
            lni = l * ni
            for r in range(NB):
                def idx_body(c, carry3):
                    off = c * 16
                    a16 = a_v[pl.ds(off, 16)]
                    b16 = b_v[r, pl.ds(off, 16)]
                    blk = ((lni + lax.shift_right_logical(a16, sh_i))
                           * nj + lax.shift_right_logical(b16, sh_j))
                    word = (blk * block_words
                            + lax.bitwise_and(a16, _TB_I - 1) * hw
                            + lax.bitwise_and(b16, hw - 1))
                    q = lax.div(c, jnp.int32(_Q // 16))
                    within = lax.rem(c, jnp.int32(_Q // 16))
                    idx_v[r * nq + q, pl.ds(within * 16, 16)] = word
                    return carry3

                lax.fori_loop(0, ch // 16, idx_body, 0)
            copies = []
            for r in range(NB):
                for q in range(nq):
                    copies.append(pltpu.async_copy(
                        g_hbm.at[idx_v.at[r * nq + q]],
                        simu_v.at[r, pl.ds(q * _Q, _Q)], sem))
            for cp in copies:
                cp.wait()
            pltpu.sync_copy(simu_v, out_hbm.at[l, :, pl.ds(base, ch)])
            return carry

        lax.fori_loop(0, L, layer_body, 0)

    return sc_gather


def _loss_body(su_ref, b_ref, o_ref):
    v = su_ref[...]
    odd = lax.bitwise_and(lax.shift_right_logical(b_ref[...], 7), 1)
    odd3 = jnp.broadcast_to(odd[None], v.shape)
    h = jnp.where(odd3 == 1,
                  lax.shift_right_logical(v, jnp.uint32(16)),
                  lax.bitwise_and(v, jnp.uint32(0xFFFF)))
    s = lax.bitcast_convert_type(
        lax.shift_left(h, jnp.uint32(16)), jnp.float32)
    logits = s * (1.0 / TEMPERATURE)
    pos = logits[:, 0, :]
    denom = jnp.exp(pos) + jnp.sum(jnp.exp(logits[:, 1:, :]), axis=1)
    terms = pos - jnp.log(denom)
    L = terms.shape[0]
    o_ref[...] = (-jnp.sum(jnp.mean(terms, axis=-1)) / L).reshape(1, 1)


def _loss(sims_u, b_all):
    return pl.pallas_call(
        _loss_body,
        out_shape=jax.ShapeDtypeStruct((1, 1), jnp.float32),
    )(sims_u, b_all)


def kernel(activations, anchor_idx, pos_idx, neg_idx):
    L, B, S, D = activations.shape
    R = B * S
    N, K = neg_idx.shape
    flat = activations.reshape(L, R, D)

    xhat = _normalize(flat)
    gram = _gram(xhat)
    g_flat = gram.reshape(L * R * R // 2)

    b_all = jnp.concatenate([pos_idx[None, :], neg_idx.T], axis=0)
    sims_u = _make_sc_gather(L, R, N, K + 1)(g_flat, anchor_idx, b_all)
    return _loss(sims_u, b_all)[0, 0]


# speedup vs baseline: 14.9428x; 1.0268x over previous
import functools

import jax
import jax.numpy as jnp
from jax import lax
from jax.experimental import pallas as pl
from jax.experimental.pallas import tpu as pltpu
from jax.experimental.pallas import tpu_sc as plsc

TEMPERATURE = 0.1
EPS = 1e-8

_NC = 2
_NS = 16
_NW = _NC * _NS

_ROW_W = 16
_Q = 128


def _normalize_body(x_ref, o_ref):
    x = x_ref[...]
    nrm = jnp.sqrt(jnp.sum(x * x, axis=-1, keepdims=True))
    o_ref[...] = (x / jnp.maximum(nrm, EPS)).astype(jnp.bfloat16)


def _normalize(flat):
    L, R, D = flat.shape
    rb = 1024
    return pl.pallas_call(
        _normalize_body,
        grid=(L, R // rb),
        in_specs=[pl.BlockSpec((1, rb, D), lambda l, i: (l, i, 0))],
        out_specs=pl.BlockSpec((1, rb, D), lambda l, i: (l, i, 0)),
        out_shape=jax.ShapeDtypeStruct((L, R, D), jnp.bfloat16),
    )(flat)


_TB_I = 1024
_TB_J = 256
_TRI = 40
_T0, _T1, _T2 = 16, 28, 36


def _gram_body(a_ref, b_ref, o_ref):
    d = lax.dot_general(
        a_ref[0], b_ref[0],
        (((1,), (1,)), ((), ())),
        preferred_element_type=jnp.float32,
    )
    lo = lax.bitcast_convert_type(
        d[:, :_TB_J // 2].astype(jnp.bfloat16), jnp.uint16)
    hi = lax.bitcast_convert_type(
        d[:, _TB_J // 2:].astype(jnp.bfloat16), jnp.uint16)
    o_ref[...] = lo.astype(jnp.uint32) | (hi.astype(jnp.uint32) << 16)


def _tri_i(t):
    return ((t >= _T0).astype(jnp.int32) + (t >= _T1).astype(jnp.int32)
            + (t >= _T2).astype(jnp.int32))


def _gram(xhat):
    L, R, D = xhat.shape

    def amap(l, t):
        return (l, _tri_i(t), 0)

    def bmap(l, t):
        i = _tri_i(t)
        return (l, t - (14 * i - 2 * i * i), 0)

    return pl.pallas_call(
        _gram_body,
        grid=(L, _TRI),
        in_specs=[
            pl.BlockSpec((1, _TB_I, D), amap),
            pl.BlockSpec((1, _TB_J, D), bmap),
        ],
        out_specs=pl.BlockSpec(
            (_TB_I, _TB_J // 2), lambda l, t: (l * _TRI + t, 0)),
        out_shape=jax.ShapeDtypeStruct((L * _TRI * _TB_I, _TB_J // 2),
                                       jnp.uint32),
    )(xhat, xhat)


def _make_sc_gather(L, R, N, NB):
    sh_i = _TB_I.bit_length() - 1
    sh_j = _TB_J.bit_length() - 1
    hw = _TB_J // 2
    block_words = _TB_I * hw
    ch = N // _NW
    nq = ch // _Q

    mesh = plsc.VectorSubcoreMesh(core_axis_name="c", subcore_axis_name="s")

    @functools.partial(
        pl.kernel,
        out_type=jax.ShapeDtypeStruct((L, NB, N), jnp.uint32),
        mesh=mesh,
        scratch_types=[
            pltpu.VMEM((ch,), jnp.int32),
            pltpu.VMEM((NB, ch), jnp.int32),
            pltpu.VMEM((NB * nq, _Q), jnp.int32),
            pltpu.VMEM((NB, ch), jnp.uint32),
            pltpu.SemaphoreType.DMA,
        ],
    )
    def sc_gather(g_hbm, a_hbm, b_hbm, out_hbm, a_v, b_v, idx_v, simu_v,
                  sem):
        wid = lax.axis_index("s") * _NC + lax.axis_index("c")
        base = wid * ch
        pltpu.sync_copy(a_hbm.at[pl.ds(base, ch)], a_v)
        for r in range(NB):
            pltpu.sync_copy(b_hbm.at[r, pl.ds(base, ch)], b_v.at[r])

        def layer_body(l, carry):
            ltri = l * _TRI
            for r in range(NB):
                def idx_body(c, carry3):
                    off = c * 16
                    a16 = a_v[pl.ds(off, 16)]
                    b16 = b_v[r, pl.ds(off, 16)]
                    swap = (lax.shift_right_logical(a16, sh_i)
                            > lax.shift_right_logical(b16, sh_i))
                    aa = jnp.where(swap, b16, a16)
                    bb = jnp.where(swap, a16, b16)
                    i = lax.shift_right_logical(aa, sh_i)
                    j = lax.shift_right_logical(bb, sh_j)
                    t = 14 * i - 2 * i * i + j
                    word = ((ltri + t) * block_words
                            + lax.bitwise_and(aa, _TB_I - 1) * hw
                            + lax.bitwise_and(bb, hw - 1))
                    q = lax.div(c, jnp.int32(_Q // 16))
                    within = lax.rem(c, jnp.int32(_Q // 16))
                    idx_v[r * nq + q, pl.ds(within * 16, 16)] = word
                    return carry3

                lax.fori_loop(0, ch // 16, idx_body, 0)
            copies = []
            for r in range(NB):
                for q in range(nq):
                    copies.append(pltpu.async_copy(
                        g_hbm.at[idx_v.at[r * nq + q]],
                        simu_v.at[r, pl.ds(q * _Q, _Q)], sem))
            for cp in copies:
                cp.wait()
            pltpu.sync_copy(simu_v, out_hbm.at[l, :, pl.ds(base, ch)])
            return carry

        lax.fori_loop(0, L, layer_body, 0)

    return sc_gather


def _loss_body(su_ref, b_ref, a_ref, o_ref):
    v = su_ref[...]
    b = b_ref[...]
    a = jnp.broadcast_to(a_ref[...][None], b.shape)
    sh = _TB_I.bit_length() - 1
    bb = jnp.where(lax.shift_right_logical(a, sh)
                   > lax.shift_right_logical(b, sh), a, b)
    odd = lax.bitwise_and(lax.shift_right_logical(bb, 7), 1)
    odd3 = jnp.broadcast_to(odd[None], v.shape)
    h = jnp.where(odd3 == 1,
                  lax.shift_right_logical(v, jnp.uint32(16)),
                  lax.bitwise_and(v, jnp.uint32(0xFFFF)))
    s = lax.bitcast_convert_type(
        lax.shift_left(h, jnp.uint32(16)), jnp.float32)
    logits = s * (1.0 / TEMPERATURE)
    pos = logits[:, 0, :]
    denom = jnp.exp(pos) + jnp.sum(jnp.exp(logits[:, 1:, :]), axis=1)
    terms = pos - jnp.log(denom)
    L = terms.shape[0]
    o_ref[...] = (-jnp.sum(jnp.mean(terms, axis=-1)) / L).reshape(1, 1)


def _loss(sims_u, b_all, anchor_idx):
    return pl.pallas_call(
        _loss_body,
        out_shape=jax.ShapeDtypeStruct((1, 1), jnp.float32),
    )(sims_u, b_all, anchor_idx)


def kernel(activations, anchor_idx, pos_idx, neg_idx):
    L, B, S, D = activations.shape
    R = B * S
    N, K = neg_idx.shape
    flat = activations.reshape(L, R, D)

    xhat = _normalize(flat)
    gram = _gram(xhat)
    g_flat = gram.reshape(L * _TRI * _TB_I * (_TB_J // 2))

    b_all = jnp.concatenate([pos_idx[None, :], neg_idx.T], axis=0)
    sims_u = _make_sc_gather(L, R, N, K + 1)(g_flat, anchor_idx, b_all)
    return _loss(sims_u, b_all, anchor_idx)[0, 0]


# speedup vs baseline: 15.5831x; 1.0429x over previous
import functools

import jax
import jax.numpy as jnp
from jax import lax
from jax.experimental import pallas as pl
from jax.experimental.pallas import tpu as pltpu
from jax.experimental.pallas import tpu_sc as plsc

TEMPERATURE = 0.1
EPS = 1e-8

_NC = 2
_NS = 16
_NW = _NC * _NS

_ROW_W = 16
_Q = 128


def _normalize_body(x_ref, o_ref):
    x = x_ref[...]
    nrm = jnp.sqrt(jnp.sum(x * x, axis=-1, keepdims=True))
    o_ref[...] = (x / jnp.maximum(nrm, EPS)).astype(jnp.bfloat16)


def _normalize(flat):
    L, R, D = flat.shape
    rb = 1024
    return pl.pallas_call(
        _normalize_body,
        grid=(L, R // rb),
        in_specs=[pl.BlockSpec((1, rb, D), lambda l, i: (l, i, 0))],
        out_specs=pl.BlockSpec((1, rb, D), lambda l, i: (l, i, 0)),
        out_shape=jax.ShapeDtypeStruct((L, R, D), jnp.bfloat16),
    )(flat)


_TB_I = 1024
_TB_J = 256
_TRI = 40
_T0, _T1, _T2 = 16, 28, 36


def _gram_body(a_ref, b_ref, o_ref):
    d = lax.dot_general(
        a_ref[0], b_ref[0],
        (((1,), (1,)), ((), ())),
        preferred_element_type=jnp.float32,
    )
    lo = lax.bitcast_convert_type(
        d[:, :_TB_J // 2].astype(jnp.bfloat16), jnp.uint16)
    hi = lax.bitcast_convert_type(
        d[:, _TB_J // 2:].astype(jnp.bfloat16), jnp.uint16)
    o_ref[...] = lo.astype(jnp.uint32) | (hi.astype(jnp.uint32) << 16)


def _tri_i(t):
    return ((t >= _T0).astype(jnp.int32) + (t >= _T1).astype(jnp.int32)
            + (t >= _T2).astype(jnp.int32))


def _gram(xhat, l0, lh):
    L, R, D = xhat.shape

    def amap(l, t):
        return (l0 + l, _tri_i(t), 0)

    def bmap(l, t):
        i = _tri_i(t)
        return (l0 + l, t - (14 * i - 2 * i * i), 0)

    return pl.pallas_call(
        _gram_body,
        grid=(lh, _TRI),
        in_specs=[
            pl.BlockSpec((1, _TB_I, D), amap),
            pl.BlockSpec((1, _TB_J, D), bmap),
        ],
        out_specs=pl.BlockSpec(
            (_TB_I, _TB_J // 2), lambda l, t: (l * _TRI + t, 0)),
        out_shape=jax.ShapeDtypeStruct((lh * _TRI * _TB_I, _TB_J // 2),
                                       jnp.uint32),
    )(xhat, xhat)


def _make_sc_gather(L, R, N, NB):
    sh_i = _TB_I.bit_length() - 1
    sh_j = _TB_J.bit_length() - 1
    hw = _TB_J // 2
    block_words = _TB_I * hw
    ch = N // _NW
    nq = ch // _Q

    mesh = plsc.VectorSubcoreMesh(core_axis_name="c", subcore_axis_name="s")

    @functools.partial(
        pl.kernel,
        out_type=jax.ShapeDtypeStruct((L, NB, N), jnp.uint32),
        mesh=mesh,
        scratch_types=[
            pltpu.VMEM((ch,), jnp.int32),
            pltpu.VMEM((NB, ch), jnp.int32),
            pltpu.VMEM((NB * nq, _Q), jnp.int32),
            pltpu.VMEM((NB, ch), jnp.uint32),
            pltpu.SemaphoreType.DMA,
        ],
    )
    def sc_gather(g_hbm, a_hbm, b_hbm, out_hbm, a_v, b_v, idx_v, simu_v,
                  sem):
        wid = lax.axis_index("s") * _NC + lax.axis_index("c")
        base = wid * ch
        pltpu.sync_copy(a_hbm.at[pl.ds(base, ch)], a_v)
        for r in range(NB):
            pltpu.sync_copy(b_hbm.at[r, pl.ds(base, ch)], b_v.at[r])

        def layer_body(l, carry):
            ltri = l * _TRI
            for r in range(NB):
                def idx_body(c, carry3):
                    off = c * 16
                    a16 = a_v[pl.ds(off, 16)]
                    b16 = b_v[r, pl.ds(off, 16)]
                    swap = (lax.shift_right_logical(a16, sh_i)
                            > lax.shift_right_logical(b16, sh_i))
                    aa = jnp.where(swap, b16, a16)
                    bb = jnp.where(swap, a16, b16)
                    i = lax.shift_right_logical(aa, sh_i)
                    j = lax.shift_right_logical(bb, sh_j)
                    t = 14 * i - 2 * i * i + j
                    word = ((ltri + t) * block_words
                            + lax.bitwise_and(aa, _TB_I - 1) * hw
                            + lax.bitwise_and(bb, hw - 1))
                    q = lax.div(c, jnp.int32(_Q // 16))
                    within = lax.rem(c, jnp.int32(_Q // 16))
                    idx_v[r * nq + q, pl.ds(within * 16, 16)] = word
                    return carry3

                lax.fori_loop(0, ch // 16, idx_body, 0)
            copies = []
            for r in range(NB):
                for q in range(nq):
                    copies.append(pltpu.async_copy(
                        g_hbm.at[idx_v.at[r * nq + q]],
                        simu_v.at[r, pl.ds(q * _Q, _Q)], sem))
            for cp in copies:
                cp.wait()
            pltpu.sync_copy(simu_v, out_hbm.at[l, :, pl.ds(base, ch)])
            return carry

        lax.fori_loop(0, L, layer_body, 0)

    return sc_gather


def _loss_body(su0_ref, su1_ref, b_ref, a_ref, o_ref):
    v = jnp.concatenate([su0_ref[...], su1_ref[...]], axis=0)
    b = b_ref[...]
    a = jnp.broadcast_to(a_ref[...][None], b.shape)
    sh = _TB_I.bit_length() - 1
    bb = jnp.where(lax.shift_right_logical(a, sh)
                   > lax.shift_right_logical(b, sh), a, b)
    odd = lax.bitwise_and(lax.shift_right_logical(bb, 7), 1)
    odd3 = jnp.broadcast_to(odd[None], v.shape)
    h = jnp.where(odd3 == 1,
                  lax.shift_right_logical(v, jnp.uint32(16)),
                  lax.bitwise_and(v, jnp.uint32(0xFFFF)))
    s = lax.bitcast_convert_type(
        lax.shift_left(h, jnp.uint32(16)), jnp.float32)
    logits = s * (1.0 / TEMPERATURE)
    pos = logits[:, 0, :]
    denom = jnp.exp(pos) + jnp.sum(jnp.exp(logits[:, 1:, :]), axis=1)
    terms = pos - jnp.log(denom)
    L = terms.shape[0]
    o_ref[...] = (-jnp.sum(jnp.mean(terms, axis=-1)) / L).reshape(1, 1)


def _loss(sims_u0, sims_u1, b_all, anchor_idx):
    return pl.pallas_call(
        _loss_body,
        out_shape=jax.ShapeDtypeStruct((1, 1), jnp.float32),
    )(sims_u0, sims_u1, b_all, anchor_idx)


def kernel(activations, anchor_idx, pos_idx, neg_idx):
    L, B, S, D = activations.shape
    R = B * S
    N, K = neg_idx.shape
    flat = activations.reshape(L, R, D)

    xhat = _normalize(flat)
    b_all = jnp.concatenate([pos_idx[None, :], neg_idx.T], axis=0)

    lh = L // 2
    sc = _make_sc_gather(lh, R, N, K + 1)
    flat_words = lh * _TRI * _TB_I * (_TB_J // 2)
    g0 = _gram(xhat, 0, lh).reshape(flat_words)
    sims_u0 = sc(g0, anchor_idx, b_all)
    g1 = _gram(xhat, lh, lh).reshape(flat_words)
    sims_u1 = sc(g1, anchor_idx, b_all)
    return _loss(sims_u0, sims_u1, b_all, anchor_idx)[0, 0]
